# baseline (speedup 1.0000x reference)
import functools

import jax
import jax.numpy as jnp
from jax.experimental import pallas as pl
from jax.experimental.pallas import tpu as pltpu

_SIM_T = 0.7
_VIRAL_T = 0.2
_K = 10
_CW = 16
_AW = 8
_NC = 2


def _pick_nt(n):
    for c in (2000, 2048, 1024, 1000, 512, 256, 128, 64, 32, 16, 8):
        if n % c == 0:
            return c
    return n


def _insert(cval, cvir, ccnt, c16i, do, v, vir_s, cnt_s):
    c = cval[...]
    mn = jnp.min(c, axis=1, keepdims=True)
    do = do & (v > mn)
    colmn = jnp.min(jnp.where(c == mn, c16i, _CW), axis=1, keepdims=True)
    upd = (c16i == colmn) & do
    cval[...] = jnp.where(upd, v, c)
    cvir[...] = jnp.where(upd, vir_s, cvir[...])
    ccnt[...] = jnp.where(upd, cnt_s, ccnt[...])


def _scan_kernel(feats_ref, keys_ref, aux_ref, viral_ref, cnt_ref,
                 oval_ref, ovir_ref, ocnt_ref,
                 s_scr, cval, cvir, ccnt, rem, *, nt, half_tiles):
    n = pl.program_id(1)
    c16i = jax.lax.broadcasted_iota(jnp.int32, cval.shape, 1)

    @pl.when(n == 0)
    def _init():
        cval[...] = jnp.where(c16i < _K, jnp.float32(-1e9), jnp.float32(1e30))
        cvir[...] = jnp.zeros_like(cvir)
        ccnt[...] = jnp.zeros_like(ccnt)

    @pl.when(n > 0)
    def _process_prev():
        s = s_scr[...]
        mask = (s > _SIM_T).astype(jnp.float32)
        aux = aux_ref[0]
        m = jax.lax.dot_general(mask, aux, (((1,), (0,)), ((), ())),
                                preferred_element_type=jnp.float32)
        count = m[:, 0:1]
        maxcnt = jnp.max(count)

        @pl.when((maxcnt > 0.5) & (maxcnt < 1.5))
        def _fast():
            v = jnp.max(s, axis=1, keepdims=True)
            _insert(cval, cvir, ccnt, c16i, count > 0.5, v,
                    m[:, 1:2], m[:, 2:3])

        @pl.when(maxcnt > 1.5)
        def _slow():
            rem[0] = maxcnt
            iota = jax.lax.broadcasted_iota(jnp.int32, s.shape, 1)
            vrow = viral_ref[0]
            crow = cnt_ref[0]
            for _ in range(_K):
                @pl.when(rem[0] > 0.5)
                def _one():
                    sk = s_scr[...]
                    mv = jnp.max(sk, axis=1, keepdims=True)
                    col = jnp.min(jnp.where(sk == mv, iota, nt),
                                  axis=1, keepdims=True)
                    onec = iota == col
                    vir_s = jnp.sum(jnp.where(onec, vrow, 0.0),
                                    axis=1, keepdims=True)
                    cnt_s = jnp.sum(jnp.where(onec, crow, 0.0),
                                    axis=1, keepdims=True)
                    smask = jnp.where(onec, jnp.float32(-1e9), sk)
                    s_scr[...] = smask
                    rem[0] = jnp.where(jnp.max(smask) > _SIM_T, 1.0, 0.0)
                    _insert(cval, cvir, ccnt, c16i, mv > _SIM_T, mv,
                            vir_s, cnt_s)

    @pl.when(n < half_tiles)
    def _matmul():
        feats = feats_ref[...]
        keys = keys_ref[...].astype(jnp.bfloat16)
        s = jax.lax.dot_general(feats, keys, (((1,), (1,)), ((), ())),
                                preferred_element_type=jnp.float32)
        s_scr[...] = s

    @pl.when(n == half_tiles)
    def _emit():
        oval_ref[0] = cval[...]
        ovir_ref[0] = cvir[...]
        ocnt_ref[0] = ccnt[...]


def _merge_kernel(val_ref, vir_ref, cnt_ref, out_ref):
    b = val_ref.shape[1]
    v2 = jnp.concatenate([val_ref[0], val_ref[1]], axis=1)
    r2 = jnp.concatenate([vir_ref[0], vir_ref[1]], axis=1)
    n2 = jnp.concatenate([cnt_ref[0], cnt_ref[1]], axis=1)
    v2 = jnp.where(v2 > jnp.float32(1e20), jnp.float32(-1e9), v2)
    w2i = jax.lax.broadcasted_iota(jnp.int32, v2.shape, 1)
    c16i = jax.lax.broadcasted_iota(jnp.int32, (b, _CW), 1)

    vals = jnp.where(c16i < _K, jnp.float32(-1e9), jnp.float32(1e30))
    vir = jnp.zeros((b, _CW), jnp.float32)
    cnt = jnp.zeros((b, _CW), jnp.float32)
    for _ in range(_K):
        mv = jnp.max(v2, axis=1, keepdims=True)
        col = jnp.min(jnp.where(v2 == mv, w2i, 2 * _CW),
                      axis=1, keepdims=True)
        onec = w2i == col
        vir_s = jnp.sum(jnp.where(onec, r2, 0.0), axis=1, keepdims=True)
        cnt_s = jnp.sum(jnp.where(onec, n2, 0.0), axis=1, keepdims=True)
        v2 = jnp.where(onec, jnp.float32(-1e9), v2)
        do = mv > _SIM_T
        mn = jnp.min(vals, axis=1, keepdims=True)
        colmn = jnp.min(jnp.where(vals == mn, c16i, _CW),
                        axis=1, keepdims=True)
        upd = (c16i == colmn) & do & (mv > mn)
        vals = jnp.where(upd, mv, vals)
        vir = jnp.where(upd, vir_s, vir)
        cnt = jnp.where(upd, cnt_s, cnt)

    keep = (vals > _SIM_T) & (c16i < _K)
    kv = keep & (vir > 0.5)
    nk = jnp.sum(keep.astype(jnp.float32), axis=1, keepdims=True)
    nv = jnp.sum(kv.astype(jnp.float32), axis=1, keepdims=True)
    mx = jnp.max(jnp.where(kv, vals, jnp.float32(-1e9)), axis=1,
                 keepdims=True)
    e = jnp.where(kv, jnp.exp(vals - mx), 0.0)
    z = jnp.sum(e, axis=1, keepdims=True)
    p = jnp.sum(e * cnt, axis=1, keepdims=True)
    pred = p / jnp.maximum(z, jnp.float32(1e-30))
    ratio = nv / jnp.maximum(nk, 1.0)
    cond = (nk > 0) & (ratio >= _VIRAL_T) & (nv > 0)
    out_ref[...] = jnp.where(cond, pred, 0.0)


@jax.jit
def kernel(feature_embedding, keys, if_viral, retweet_cnt):
    b, d = feature_embedding.shape
    n = keys.shape[0]
    nt = _pick_nt(n)
    n_tiles = n // nt
    nc = _NC if n_tiles % _NC == 0 else 1
    half_tiles = n_tiles // nc
    viral_f = if_viral.astype(jnp.float32)
    cnt_f = retweet_cnt.astype(jnp.float32)
    aux = jnp.concatenate(
        [jnp.ones((n, 1), jnp.float32), viral_f[:, None], cnt_f[:, None],
         jnp.zeros((n, _AW - 3), jnp.float32)], axis=1,
    ).reshape(n_tiles, nt, _AW)
    viral3d = viral_f.reshape(n_tiles, 1, nt)
    cnt3d = cnt_f.reshape(n_tiles, 1, nt)

    cvals, cvirs, ccnts = pl.pallas_call(
        functools.partial(_scan_kernel, nt=nt, half_tiles=half_tiles),
        grid=(nc, half_tiles + 1),
        in_specs=[
            pl.BlockSpec((b, d), lambda c, i: (0, 0)),
            pl.BlockSpec(
                (nt, d),
                lambda c, i: (c * half_tiles + jnp.minimum(i, half_tiles - 1),
                              0)),
            pl.BlockSpec(
                (1, nt, _AW),
                lambda c, i: (c * half_tiles + jnp.maximum(i - 1, 0), 0, 0)),
            pl.BlockSpec(
                (1, 1, nt),
                lambda c, i: (c * half_tiles + jnp.maximum(i - 1, 0), 0, 0)),
            pl.BlockSpec(
                (1, 1, nt),
                lambda c, i: (c * half_tiles + jnp.maximum(i - 1, 0), 0, 0)),
        ],
        out_specs=[
            pl.BlockSpec((1, b, _CW), lambda c, i: (c, 0, 0)),
            pl.BlockSpec((1, b, _CW), lambda c, i: (c, 0, 0)),
            pl.BlockSpec((1, b, _CW), lambda c, i: (c, 0, 0)),
        ],
        out_shape=[
            jax.ShapeDtypeStruct((nc, b, _CW), jnp.float32),
            jax.ShapeDtypeStruct((nc, b, _CW), jnp.float32),
            jax.ShapeDtypeStruct((nc, b, _CW), jnp.float32),
        ],
        scratch_shapes=[
            pltpu.VMEM((b, nt), jnp.float32),
            pltpu.VMEM((b, _CW), jnp.float32),
            pltpu.VMEM((b, _CW), jnp.float32),
            pltpu.VMEM((b, _CW), jnp.float32),
            pltpu.SMEM((1,), jnp.float32),
        ],
        compiler_params=pltpu.CompilerParams(
            dimension_semantics=("parallel", "arbitrary"),
            vmem_limit_bytes=112 * 1024 * 1024,
        ),
    )(feature_embedding.astype(jnp.bfloat16), keys, aux, viral3d, cnt3d)

    if nc == 1:
        empty = jnp.full_like(cvals, -1e9)
        cvals = jnp.concatenate([cvals, empty], axis=0)
        cvirs = jnp.concatenate([cvirs, jnp.zeros_like(cvirs)], axis=0)
        ccnts = jnp.concatenate([ccnts, jnp.zeros_like(ccnts)], axis=0)

    out = pl.pallas_call(
        _merge_kernel,
        out_shape=jax.ShapeDtypeStruct((b, 1), jnp.float32),
    )(cvals, cvirs, ccnts)
    return out.reshape(b)


# speedup vs baseline: 1.0199x; 1.0199x over previous
import functools

import jax
import jax.numpy as jnp
from jax.experimental import pallas as pl
from jax.experimental.pallas import tpu as pltpu

_SIM_T = 0.7
_VIRAL_T = 0.2
_K = 10
_CW = 16
_AW = 8


def _pick_nt(n):
    for c in (1000, 1024, 512, 256, 128, 64, 32, 16, 8):
        if n % (2 * c) == 0:
            return c
    return None


def _insert(cval, cvir, ccnt, c16i, do, v, vir_s, cnt_s):
    c = cval[...]
    mn = jnp.min(c, axis=1, keepdims=True)
    do = do & (v > mn)
    colmn = jnp.min(jnp.where(c == mn, c16i, _CW), axis=1, keepdims=True)
    upd = (c16i == colmn) & do
    cval[...] = jnp.where(upd, v, c)
    cvir[...] = jnp.where(upd, vir_s, cvir[...])
    ccnt[...] = jnp.where(upd, cnt_s, ccnt[...])


def _knn_kernel(feats_ref, keysa_ref, keysb_ref, aux_ref, viral_ref, cnt_ref,
                out_ref, s_scr, cval, cvir, ccnt, rem, *, nt, n_steps):
    n = pl.program_id(0)
    nt2 = 2 * nt
    c16i = jax.lax.broadcasted_iota(jnp.int32, cval.shape, 1)

    @pl.when(n == 0)
    def _init():
        cval[...] = jnp.where(c16i < _K, jnp.float32(-1e9), jnp.float32(1e30))
        cvir[...] = jnp.zeros_like(cvir)
        ccnt[...] = jnp.zeros_like(ccnt)

    @pl.when(n > 0)
    def _process_prev():
        s = s_scr[...]
        mask = (s > _SIM_T).astype(jnp.float32)
        aux = aux_ref[...].reshape(nt2, _AW)
        m = jax.lax.dot_general(mask, aux, (((1,), (0,)), ((), ())),
                                preferred_element_type=jnp.float32)
        count = m[:, 0:1]
        maxcnt = jnp.max(count)

        @pl.when((maxcnt > 0.5) & (maxcnt < 1.5))
        def _fast():
            v = jnp.max(s, axis=1, keepdims=True)
            _insert(cval, cvir, ccnt, c16i, count > 0.5, v,
                    m[:, 1:2], m[:, 2:3])

        @pl.when(maxcnt > 1.5)
        def _slow():
            rem[0] = maxcnt
            iota = jax.lax.broadcasted_iota(jnp.int32, s.shape, 1)
            vrow = viral_ref[...].reshape(1, nt2)
            crow = cnt_ref[...].reshape(1, nt2)
            for _ in range(_K):
                @pl.when(rem[0] > 0.5)
                def _one():
                    sk = s_scr[...]
                    mv = jnp.max(sk, axis=1, keepdims=True)
                    col = jnp.min(jnp.where(sk == mv, iota, nt2),
                                  axis=1, keepdims=True)
                    onec = iota == col
                    vir_s = jnp.sum(jnp.where(onec, vrow, 0.0),
                                    axis=1, keepdims=True)
                    cnt_s = jnp.sum(jnp.where(onec, crow, 0.0),
                                    axis=1, keepdims=True)
                    smask = jnp.where(onec, jnp.float32(-1e9), sk)
                    s_scr[...] = smask
                    rem[0] = jnp.where(jnp.max(smask) > _SIM_T, 1.0, 0.0)
                    _insert(cval, cvir, ccnt, c16i, mv > _SIM_T, mv,
                            vir_s, cnt_s)

    @pl.when(n < n_steps)
    def _matmul():
        feats = feats_ref[...]
        ka = keysa_ref[0].astype(jnp.bfloat16)
        kb = keysb_ref[0].astype(jnp.bfloat16)
        sa = jax.lax.dot_general(feats, ka, (((1,), (1,)), ((), ())),
                                 preferred_element_type=jnp.float32)
        sb = jax.lax.dot_general(feats, kb, (((1,), (1,)), ((), ())),
                                 preferred_element_type=jnp.float32)
        s_scr[:, :nt] = sa
        s_scr[:, nt:] = sb

    @pl.when(n == n_steps)
    def _finalize():
        vals = cval[...]
        vir = cvir[...]
        cnt = ccnt[...]
        keep = (vals > _SIM_T) & (c16i < _K)
        kv = keep & (vir > 0.5)
        nk = jnp.sum(keep.astype(jnp.float32), axis=1, keepdims=True)
        nv = jnp.sum(kv.astype(jnp.float32), axis=1, keepdims=True)
        mx = jnp.max(jnp.where(kv, vals, jnp.float32(-1e9)),
                     axis=1, keepdims=True)
        e = jnp.where(kv, jnp.exp(vals - mx), 0.0)
        z = jnp.sum(e, axis=1, keepdims=True)
        p = jnp.sum(e * cnt, axis=1, keepdims=True)
        pred = p / jnp.maximum(z, jnp.float32(1e-30))
        ratio = nv / jnp.maximum(nk, 1.0)
        cond = (nk > 0) & (ratio >= _VIRAL_T) & (nv > 0)
        out_ref[...] = jnp.where(cond, pred, 0.0)


@jax.jit
def kernel(feature_embedding, keys, if_viral, retweet_cnt):
    b, d = feature_embedding.shape
    n = keys.shape[0]
    nt = _pick_nt(n)
    n_tiles = n // nt
    n_steps = n_tiles // 2
    viral_f = if_viral.astype(jnp.float32)
    cnt_f = retweet_cnt.astype(jnp.float32)
    aux = jnp.concatenate(
        [jnp.ones((n, 1), jnp.float32), viral_f[:, None], cnt_f[:, None],
         jnp.zeros((n, _AW - 3), jnp.float32)], axis=1,
    ).reshape(n_steps, 2 * nt, _AW)
    viral3d = viral_f.reshape(n_steps, 1, 2 * nt)
    cnt3d = cnt_f.reshape(n_steps, 1, 2 * nt)
    keys3d = keys.reshape(n_tiles, nt, d)

    last = n_steps - 1
    out = pl.pallas_call(
        functools.partial(_knn_kernel, nt=nt, n_steps=n_steps),
        grid=(n_steps + 1,),
        in_specs=[
            pl.BlockSpec((b, d), lambda i: (0, 0)),
            pl.BlockSpec((1, nt, d),
                         lambda i: (2 * jnp.minimum(i, last), 0, 0)),
            pl.BlockSpec((1, nt, d),
                         lambda i: (2 * jnp.minimum(i, last) + 1, 0, 0)),
            pl.BlockSpec((1, 2 * nt, _AW),
                         lambda i: (jnp.maximum(i - 1, 0), 0, 0)),
            pl.BlockSpec((1, 1, 2 * nt),
                         lambda i: (jnp.maximum(i - 1, 0), 0, 0)),
            pl.BlockSpec((1, 1, 2 * nt),
                         lambda i: (jnp.maximum(i - 1, 0), 0, 0)),
        ],
        out_specs=pl.BlockSpec((b, 1), lambda i: (0, 0)),
        out_shape=jax.ShapeDtypeStruct((b, 1), jnp.float32),
        scratch_shapes=[
            pltpu.VMEM((b, 2 * nt), jnp.float32),
            pltpu.VMEM((b, _CW), jnp.float32),
            pltpu.VMEM((b, _CW), jnp.float32),
            pltpu.VMEM((b, _CW), jnp.float32),
            pltpu.SMEM((1,), jnp.float32),
        ],
        compiler_params=pltpu.CompilerParams(
            dimension_semantics=("arbitrary",),
            vmem_limit_bytes=63 * 1024 * 1024,
        ),
    )(feature_embedding.astype(jnp.bfloat16), keys3d, keys3d,
      aux, viral3d, cnt3d)
    return out.reshape(b)
